# Initial kernel scaffold; baseline (speedup 1.0000x reference)
import jax
import jax.numpy as jnp
from jax.experimental import pallas as pl


def kernel(x, edge_index, edge_attr, estimate, params):
    raise NotImplementedError("write your pallas kernel here")


# speedup vs baseline: 3.3370x; 3.3370x over previous
import functools

import jax
import jax.numpy as jnp
from jax import lax
from jax.experimental import pallas as pl
from jax.experimental.pallas import tpu as pltpu
from jax.experimental.pallas import tpu_sc as plsc

H = 128
NC, NS = 2, 16
NW = NC * NS
CH = 128
DW = 128

BN = 2000
BE = 4000


def _ln(h, g, be):
    mu = jnp.mean(h, axis=1, keepdims=True)
    var = jnp.mean((h - mu) ** 2, axis=1, keepdims=True)
    return (h - mu) / jnp.sqrt(var + 1e-5) * g + be


def _dot(a, b):
    return jnp.dot(a, b, preferred_element_type=jnp.float32)


def _enc_node_body(x_ref, est_ref, w1x, w1e, b1, w2, b2, g, be,
                   w1s, w1d, b1e, hn_ref, t_ref):
    t = _dot(x_ref[...], w1x[...]) + _dot(est_ref[...], w1e[...]) + b1[...]
    t = jnp.maximum(t, 0.0)
    h = _dot(t, w2[...]) + b2[...]
    hn = _ln(h, g[...], be[...])
    hn_ref[...] = hn
    t_ref[0] = _dot(hn, w1s[...])
    t_ref[1] = _dot(hn, w1d[...]) + b1e[...]


def _enc_edge_body(ea_ref, w1, b1, w2, b2, g, be, he_ref):
    t = _dot(ea_ref[...], w1[...]) + b1[...]
    t = jnp.maximum(t, 0.0)
    h = _dot(t, w2[...]) + b2[...]
    he_ref[...] = _ln(h, g[...], be[...])


def _edge_body(he_ref, g1_ref, g2_ref, w1e, w2, b2, g, be, out_ref):
    t = _dot(he_ref[...], w1e[...]) + g1_ref[...] + g2_ref[...]
    t = jnp.maximum(t, 0.0)
    h = _dot(t, w2[...]) + b2[...]
    out_ref[...] = he_ref[...] + _ln(h, g[...], be[...])


def _node_body(hn_ref, s_ref, d_ref, v1n, v1a, b1, v2, b2, g, be,
               w1s, w1d, b1e, hn_out, t_ref):
    deg = d_ref[0][:, :1] + d_ref[1][:, :1]
    deg = jnp.maximum(deg, 1.0)
    agg = (s_ref[0] + s_ref[1]) / deg
    t = _dot(hn_ref[...], v1n[...]) + _dot(agg, v1a[...]) + b1[...]
    t = jnp.maximum(t, 0.0)
    h = _dot(t, v2[...]) + b2[...]
    hn = hn_ref[...] + _ln(h, g[...], be[...])
    hn_out[...] = hn
    t_ref[0] = _dot(hn, w1s[...])
    t_ref[1] = _dot(hn, w1d[...]) + b1e[...]


def _dec_body(hn_ref, est_ref, w1, b1, w2, b2, out_ref):
    t = _dot(hn_ref[...], w1[...]) + b1[...]
    t = jnp.maximum(t, 0.0)
    out_ref[...] = _dot(t, w2[...]) + b2[...] + est_ref[...]


def _wspec(r, c):
    return pl.BlockSpec((r, c), lambda i: (0, 0))


def _sc_mesh():
    return plsc.VectorSubcoreMesh(core_axis_name="c", subcore_axis_name="s",
                                  num_cores=NC, num_subcores=NS)


def _chunk_counts(wid, nchunk):
    base = nchunk // NW
    rem = nchunk - base * NW
    return base + jnp.where(wid < rem, 1, 0)


def _sc_gather_body(nchunk, t_ref, src_ref, dstn_ref, g1_ref, g2_ref,
                    idx_a, idx_b, rows_a, rows_b, sem_a, sem_b):
    wid = lax.axis_index("s") * NC + lax.axis_index("c")
    n_mine = _chunk_counts(wid, nchunk)

    def body(i, carry):
        chunk = wid + i * NW
        off = chunk * CH
        pltpu.sync_copy(src_ref.at[pl.ds(off, CH)], idx_a)
        pltpu.sync_copy(dstn_ref.at[pl.ds(off, CH)], idx_b)
        cp_a = pltpu.async_copy(t_ref.at[idx_a], rows_a, sem_a)
        cp_b = pltpu.async_copy(t_ref.at[idx_b], rows_b, sem_b)
        cp_a.wait()
        cp_b.wait()
        pltpu.sync_copy(rows_a, g1_ref.at[pl.ds(off, CH)])
        pltpu.sync_copy(rows_b, g2_ref.at[pl.ds(off, CH)])
        return carry

    lax.fori_loop(0, n_mine, body, 0)


def _sc_scatter_body(rows_per_tile, nchunk, he_ref, dst_ref, zeros_ref,
                     out_ref, idx_v, rows_v, acc, sem):
    cid = lax.axis_index("c")
    sid = lax.axis_index("s")
    wid = sid * NC + cid
    r0 = sid * rows_per_tile
    pltpu.sync_copy(zeros_ref, acc.at[pl.ds(r0, rows_per_tile)])
    plsc.subcore_barrier()
    n_mine = _chunk_counts(wid, nchunk)

    def body(i, carry):
        chunk = wid + i * NW
        off = chunk * CH
        pltpu.sync_copy(dst_ref.at[pl.ds(off, CH)], idx_v)
        pltpu.sync_copy(he_ref.at[pl.ds(off, CH)], rows_v)
        pltpu.sync_copy(rows_v, acc.at[idx_v], add=True)
        return carry

    lax.fori_loop(0, n_mine, body, 0)
    plsc.subcore_barrier()
    pltpu.sync_copy(acc.at[pl.ds(r0, rows_per_tile)],
                    out_ref.at[cid, pl.ds(r0, rows_per_tile)])


def _sc_deg_body(rows_per_tile, nchunk, dst_ref, ones_ref, zeros_ref,
                 out_ref, idx_v, ones_v, acc, sem):
    cid = lax.axis_index("c")
    sid = lax.axis_index("s")
    wid = sid * NC + cid
    r0 = sid * rows_per_tile
    pltpu.sync_copy(zeros_ref, acc.at[pl.ds(r0, rows_per_tile)])
    pltpu.sync_copy(ones_ref, ones_v)
    plsc.subcore_barrier()
    n_mine = _chunk_counts(wid, nchunk)

    def body(i, carry):
        chunk = wid + i * NW
        off = chunk * CH
        pltpu.sync_copy(dst_ref.at[pl.ds(off, CH)], idx_v)
        pltpu.sync_copy(ones_v, acc.at[idx_v], add=True)
        return carry

    lax.fori_loop(0, n_mine, body, 0)
    plsc.subcore_barrier()
    pltpu.sync_copy(acc.at[pl.ds(r0, rows_per_tile)],
                    out_ref.at[cid, pl.ds(r0, rows_per_tile)])


def kernel(x, edge_index, edge_attr, estimate, params):
    n = x.shape[0]
    e = edge_index.shape[1]
    d_out = estimate.shape[1]
    d_edge = edge_attr.shape[1]
    assert e % CH == 0 and e % BE == 0 and n % BN == 0
    nchunk = e // CH
    rows_per_tile = (n + NS * 8 - 1) // (NS * 8) * 8
    n_pad = rows_per_tile * NS
    grid_n = n // BN
    grid_e = e // BE

    src = edge_index[0]
    dst = edge_index[1]
    dstn = dst + n

    p_enc_n = params['enc_n']
    p_enc_e = params['enc_e']
    p_dec = params['dec']
    procs = params['proc']
    nlayers = len(procs)

    def r1(v):
        return v.reshape(1, -1)

    ew = []
    for lp in procs:
        w1 = lp['edge']['W1']
        ew.append(dict(w1e=w1[:H], w1s=w1[H:2 * H], w1d=w1[2 * H:],
                       b1=r1(lp['edge']['b1']), w2=lp['edge']['W2'],
                       b2=r1(lp['edge']['b2']), g=r1(lp['edge']['g']),
                       be=r1(lp['edge']['be'])))
    nw_ = []
    for lp in procs:
        w1 = lp['node']['W1']
        nw_.append(dict(v1n=w1[:H], v1a=w1[H:], b1=r1(lp['node']['b1']),
                        v2=lp['node']['W2'], b2=r1(lp['node']['b2']),
                        g=r1(lp['node']['g']), be=r1(lp['node']['be'])))

    f32 = jnp.float32

    enc_n_call = pl.pallas_call(
        _enc_node_body,
        grid=(grid_n,),
        in_specs=[
            pl.BlockSpec((BN, H), lambda i: (i, 0)),
            pl.BlockSpec((BN, d_out), lambda i: (i, 0)),
            _wspec(H, H), _wspec(d_out, H), _wspec(1, H),
            _wspec(H, H), _wspec(1, H), _wspec(1, H), _wspec(1, H),
            _wspec(H, H), _wspec(H, H), _wspec(1, H),
        ],
        out_specs=[
            pl.BlockSpec((BN, H), lambda i: (i, 0)),
            pl.BlockSpec((2, BN, H), lambda i: (0, i, 0)),
        ],
        out_shape=[
            jax.ShapeDtypeStruct((n, H), f32),
            jax.ShapeDtypeStruct((2, n, H), f32),
        ],
    )
    w1n = p_enc_n['W1']
    hn, tbl = enc_n_call(x, estimate, w1n[:H], w1n[H:], r1(p_enc_n['b1']),
                         p_enc_n['W2'], r1(p_enc_n['b2']), r1(p_enc_n['g']),
                         r1(p_enc_n['be']), ew[0]['w1s'], ew[0]['w1d'],
                         ew[0]['b1'])

    enc_e_call = pl.pallas_call(
        _enc_edge_body,
        grid=(grid_e,),
        in_specs=[
            pl.BlockSpec((BE, d_edge), lambda i: (i, 0)),
            _wspec(d_edge, H), _wspec(1, H), _wspec(H, H),
            _wspec(1, H), _wspec(1, H), _wspec(1, H),
        ],
        out_specs=pl.BlockSpec((BE, H), lambda i: (i, 0)),
        out_shape=jax.ShapeDtypeStruct((e, H), f32),
    )
    he = enc_e_call(edge_attr, p_enc_e['W1'], r1(p_enc_e['b1']),
                    p_enc_e['W2'], r1(p_enc_e['b2']), r1(p_enc_e['g']),
                    r1(p_enc_e['be']))

    deg_call = pl.kernel(
        functools.partial(_sc_deg_body, rows_per_tile, nchunk),
        out_type=jax.ShapeDtypeStruct((NC, n_pad, DW), f32),
        mesh=_sc_mesh(),
        scratch_types=[
            pltpu.VMEM((CH,), jnp.int32),
            pltpu.VMEM((CH, DW), f32),
            pltpu.VMEM_SHARED((n_pad, DW), f32),
            pltpu.SemaphoreType.DMA,
        ],
    )
    degp = deg_call(dst, jnp.ones((CH, DW), f32),
                    jnp.zeros((rows_per_tile, DW), f32))

    gather_call = pl.kernel(
        functools.partial(_sc_gather_body, nchunk),
        out_type=(jax.ShapeDtypeStruct((e, H), f32),
                  jax.ShapeDtypeStruct((e, H), f32)),
        mesh=_sc_mesh(),
        scratch_types=[
            pltpu.VMEM((CH,), jnp.int32),
            pltpu.VMEM((CH,), jnp.int32),
            pltpu.VMEM((CH, H), f32),
            pltpu.VMEM((CH, H), f32),
            pltpu.SemaphoreType.DMA,
            pltpu.SemaphoreType.DMA,
        ],
    )

    scatter_call = pl.kernel(
        functools.partial(_sc_scatter_body, rows_per_tile, nchunk),
        out_type=jax.ShapeDtypeStruct((NC, n_pad, H), f32),
        mesh=_sc_mesh(),
        scratch_types=[
            pltpu.VMEM((CH,), jnp.int32),
            pltpu.VMEM((CH, H), f32),
            pltpu.VMEM_SHARED((n_pad, H), f32),
            pltpu.SemaphoreType.DMA,
        ],
    )
    zeros_nh = jnp.zeros((rows_per_tile, H), f32)

    edge_call = pl.pallas_call(
        _edge_body,
        grid=(grid_e,),
        in_specs=[
            pl.BlockSpec((BE, H), lambda i: (i, 0)),
            pl.BlockSpec((BE, H), lambda i: (i, 0)),
            pl.BlockSpec((BE, H), lambda i: (i, 0)),
            _wspec(H, H), _wspec(H, H), _wspec(1, H),
            _wspec(1, H), _wspec(1, H),
        ],
        out_specs=pl.BlockSpec((BE, H), lambda i: (i, 0)),
        out_shape=jax.ShapeDtypeStruct((e, H), f32),
    )
    node_call = pl.pallas_call(
        _node_body,
        grid=(grid_n,),
        in_specs=[
            pl.BlockSpec((BN, H), lambda i: (i, 0)),
            pl.BlockSpec((2, BN, H), lambda i: (0, i, 0)),
            pl.BlockSpec((2, BN, DW), lambda i: (0, i, 0)),
            _wspec(H, H), _wspec(H, H), _wspec(1, H),
            _wspec(H, H), _wspec(1, H), _wspec(1, H), _wspec(1, H),
            _wspec(H, H), _wspec(H, H), _wspec(1, H),
        ],
        out_specs=[
            pl.BlockSpec((BN, H), lambda i: (i, 0)),
            pl.BlockSpec((2, BN, H), lambda i: (0, i, 0)),
        ],
        out_shape=[
            jax.ShapeDtypeStruct((n, H), f32),
            jax.ShapeDtypeStruct((2, n, H), f32),
        ],
    )

    zero_w = jnp.zeros((H, H), f32)
    zero_b = jnp.zeros((1, H), f32)
    for l in range(nlayers):
        g1, g2 = gather_call(tbl.reshape(2 * n, H), src, dstn)
        ewl = ew[l]
        he = edge_call(he, g1, g2, ewl['w1e'], ewl['w2'], ewl['b2'],
                       ewl['g'], ewl['be'])
        s = scatter_call(he, dst, zeros_nh)
        nwl = nw_[l]
        if l + 1 < nlayers:
            w1s_n, w1d_n, b1_n = (ew[l + 1]['w1s'], ew[l + 1]['w1d'],
                                  ew[l + 1]['b1'])
        else:
            w1s_n, w1d_n, b1_n = zero_w, zero_w, zero_b
        hn, tbl = node_call(hn, s, degp, nwl['v1n'], nwl['v1a'], nwl['b1'],
                            nwl['v2'], nwl['b2'], nwl['g'], nwl['be'],
                            w1s_n, w1d_n, b1_n)

    dec_call = pl.pallas_call(
        _dec_body,
        grid=(grid_n,),
        in_specs=[
            pl.BlockSpec((BN, H), lambda i: (i, 0)),
            pl.BlockSpec((BN, d_out), lambda i: (i, 0)),
            _wspec(H, H), _wspec(1, H), _wspec(H, d_out), _wspec(1, d_out),
        ],
        out_specs=pl.BlockSpec((BN, d_out), lambda i: (i, 0)),
        out_shape=jax.ShapeDtypeStruct((n, d_out), f32),
    )
    out = dec_call(hn, estimate, p_dec['W1'], r1(p_dec['b1']),
                   p_dec['W2'], r1(p_dec['b2']))
    return out


# speedup vs baseline: 4.1359x; 1.2394x over previous
import functools

import jax
import jax.numpy as jnp
from jax import lax
from jax.experimental import pallas as pl
from jax.experimental.pallas import tpu as pltpu
from jax.experimental.pallas import tpu_sc as plsc

H = 128
NC, NS = 2, 16
NW = NC * NS
CH = 128
DW = 128

BN = 2000
BE = 4000


def _ln(h, g, be):
    mu = jnp.mean(h, axis=1, keepdims=True)
    var = jnp.mean((h - mu) ** 2, axis=1, keepdims=True)
    return (h - mu) / jnp.sqrt(var + 1e-5) * g + be


def _dot(a, b):
    return jnp.dot(a, b, preferred_element_type=jnp.float32)


def _enc_node_body(x_ref, est_ref, w1x, w1e, b1, w2, b2, g, be,
                   w1s, w1d, b1e, hn_ref, t_ref):
    t = _dot(x_ref[...], w1x[...]) + _dot(est_ref[...], w1e[...]) + b1[...]
    t = jnp.maximum(t, 0.0)
    h = _dot(t, w2[...]) + b2[...]
    hn = _ln(h, g[...], be[...])
    hn_ref[...] = hn
    t_ref[0] = _dot(hn, w1s[...])
    t_ref[1] = _dot(hn, w1d[...]) + b1e[...]


def _enc_edge_body(ea_ref, w1, b1, w2, b2, g, be, he_ref):
    t = _dot(ea_ref[...], w1[...]) + b1[...]
    t = jnp.maximum(t, 0.0)
    h = _dot(t, w2[...]) + b2[...]
    he_ref[...] = _ln(h, g[...], be[...])


def _edge_body(he_ref, g1_ref, g2_ref, w1e, w2, b2, g, be, out_ref):
    t = _dot(he_ref[...], w1e[...]) + g1_ref[...] + g2_ref[...]
    t = jnp.maximum(t, 0.0)
    h = _dot(t, w2[...]) + b2[...]
    out_ref[...] = he_ref[...] + _ln(h, g[...], be[...])


def _node_body(hn_ref, s_ref, d_ref, v1n, v1a, b1, v2, b2, g, be,
               w1s, w1d, b1e, hn_out, t_ref):
    deg = d_ref[0][:, :1] + d_ref[1][:, :1]
    deg = jnp.maximum(deg, 1.0)
    agg = (s_ref[0] + s_ref[1]) / deg
    t = _dot(hn_ref[...], v1n[...]) + _dot(agg, v1a[...]) + b1[...]
    t = jnp.maximum(t, 0.0)
    h = _dot(t, v2[...]) + b2[...]
    hn = hn_ref[...] + _ln(h, g[...], be[...])
    hn_out[...] = hn
    t_ref[0] = _dot(hn, w1s[...])
    t_ref[1] = _dot(hn, w1d[...]) + b1e[...]


def _dec_body(hn_ref, est_ref, w1, b1, w2, b2, out_ref):
    t = _dot(hn_ref[...], w1[...]) + b1[...]
    t = jnp.maximum(t, 0.0)
    out_ref[...] = _dot(t, w2[...]) + b2[...] + est_ref[...]


def _wspec(r, c):
    return pl.BlockSpec((r, c), lambda i: (0, 0))


def _sc_mesh():
    return plsc.VectorSubcoreMesh(core_axis_name="c", subcore_axis_name="s",
                                  num_cores=NC, num_subcores=NS)


def _chunk_counts(wid, nchunk):
    base = nchunk // NW
    rem = nchunk - base * NW
    return base + jnp.where(wid < rem, 1, 0)


K = 3


def _sc_gather_body(nchunk, t_ref, src_ref, dstn_ref, g1_ref, g2_ref,
                    idx_a, idx_b, rows_a, rows_b, sem_i, sem_g, sem_w):
    wid = lax.axis_index("s") * NC + lax.axis_index("c")
    n_mine = _chunk_counts(wid, nchunk)
    n_full = n_mine // K

    def one(chunk):
        off = chunk * CH
        pltpu.sync_copy(src_ref.at[pl.ds(off, CH)], idx_a.at[0])
        pltpu.sync_copy(dstn_ref.at[pl.ds(off, CH)], idx_b.at[0])
        cp_a = pltpu.async_copy(t_ref.at[idx_a.at[0]], rows_a.at[0], sem_g)
        cp_b = pltpu.async_copy(t_ref.at[idx_b.at[0]], rows_b.at[0], sem_g)
        cp_a.wait()
        cp_b.wait()
        pltpu.sync_copy(rows_a.at[0], g1_ref.at[pl.ds(off, CH)])
        pltpu.sync_copy(rows_b.at[0], g2_ref.at[pl.ds(off, CH)])

    def group(j, carry):
        offs = [(wid + (K * j + k) * NW) * CH for k in range(K)]
        cps = []
        for k in range(K):
            cps.append(pltpu.async_copy(
                src_ref.at[pl.ds(offs[k], CH)], idx_a.at[k], sem_i))
            cps.append(pltpu.async_copy(
                dstn_ref.at[pl.ds(offs[k], CH)], idx_b.at[k], sem_i))
        for cp in cps:
            cp.wait()
        cps = []
        for k in range(K):
            cps.append(pltpu.async_copy(
                t_ref.at[idx_a.at[k]], rows_a.at[k], sem_g))
            cps.append(pltpu.async_copy(
                t_ref.at[idx_b.at[k]], rows_b.at[k], sem_g))
        for cp in cps:
            cp.wait()
        cps = []
        for k in range(K):
            cps.append(pltpu.async_copy(
                rows_a.at[k], g1_ref.at[pl.ds(offs[k], CH)], sem_w))
            cps.append(pltpu.async_copy(
                rows_b.at[k], g2_ref.at[pl.ds(offs[k], CH)], sem_w))
        for cp in cps:
            cp.wait()
        return carry

    lax.fori_loop(0, n_full, group, 0)

    def tail(i, carry):
        one(wid + i * NW)
        return carry

    lax.fori_loop(K * n_full, n_mine, tail, 0)


def _sc_scatter_body(rows_per_tile, nchunk, he_ref, dst_ref, zeros_ref,
                     out_ref, idx_v, rows_v, acc, sem):
    cid = lax.axis_index("c")
    sid = lax.axis_index("s")
    wid = sid * NC + cid
    r0 = sid * rows_per_tile
    pltpu.sync_copy(zeros_ref, acc.at[pl.ds(r0, rows_per_tile)])
    plsc.subcore_barrier()
    n_mine = _chunk_counts(wid, nchunk)
    n_full = n_mine // K

    def group(j, carry):
        offs = [(wid + (K * j + k) * NW) * CH for k in range(K)]
        cps = []
        for k in range(K):
            cps.append(pltpu.async_copy(
                dst_ref.at[pl.ds(offs[k], CH)], idx_v.at[k], sem))
            cps.append(pltpu.async_copy(
                he_ref.at[pl.ds(offs[k], CH)], rows_v.at[k], sem))
        for cp in cps:
            cp.wait()
        cps = []
        for k in range(K):
            cps.append(pltpu.async_copy(
                rows_v.at[k], acc.at[idx_v.at[k]], sem, add=True))
        for cp in cps:
            cp.wait()
        return carry

    lax.fori_loop(0, n_full, group, 0)

    def tail(i, carry):
        off = (wid + i * NW) * CH
        pltpu.sync_copy(dst_ref.at[pl.ds(off, CH)], idx_v.at[0])
        pltpu.sync_copy(he_ref.at[pl.ds(off, CH)], rows_v.at[0])
        pltpu.sync_copy(rows_v.at[0], acc.at[idx_v.at[0]], add=True)
        return carry

    lax.fori_loop(K * n_full, n_mine, tail, 0)
    plsc.subcore_barrier()
    pltpu.sync_copy(acc.at[pl.ds(r0, rows_per_tile)],
                    out_ref.at[cid, pl.ds(r0, rows_per_tile)])


def _sc_deg_body(rows_per_tile, nchunk, dst_ref, ones_ref, zeros_ref,
                 out_ref, idx_v, ones_v, acc, sem):
    cid = lax.axis_index("c")
    sid = lax.axis_index("s")
    wid = sid * NC + cid
    r0 = sid * rows_per_tile
    pltpu.sync_copy(zeros_ref, acc.at[pl.ds(r0, rows_per_tile)])
    pltpu.sync_copy(ones_ref, ones_v)
    plsc.subcore_barrier()
    n_mine = _chunk_counts(wid, nchunk)

    def body(i, carry):
        chunk = wid + i * NW
        off = chunk * CH
        pltpu.sync_copy(dst_ref.at[pl.ds(off, CH)], idx_v)
        pltpu.sync_copy(ones_v, acc.at[idx_v], add=True)
        return carry

    lax.fori_loop(0, n_mine, body, 0)
    plsc.subcore_barrier()
    pltpu.sync_copy(acc.at[pl.ds(r0, rows_per_tile)],
                    out_ref.at[cid, pl.ds(r0, rows_per_tile)])


def kernel(x, edge_index, edge_attr, estimate, params):
    n = x.shape[0]
    e = edge_index.shape[1]
    d_out = estimate.shape[1]
    d_edge = edge_attr.shape[1]
    assert e % CH == 0 and e % BE == 0 and n % BN == 0
    nchunk = e // CH
    rows_per_tile = (n + NS * 8 - 1) // (NS * 8) * 8
    n_pad = rows_per_tile * NS
    grid_n = n // BN
    grid_e = e // BE

    src = edge_index[0]
    dst = edge_index[1]
    dstn = dst + n

    p_enc_n = params['enc_n']
    p_enc_e = params['enc_e']
    p_dec = params['dec']
    procs = params['proc']
    nlayers = len(procs)

    def r1(v):
        return v.reshape(1, -1)

    ew = []
    for lp in procs:
        w1 = lp['edge']['W1']
        ew.append(dict(w1e=w1[:H], w1s=w1[H:2 * H], w1d=w1[2 * H:],
                       b1=r1(lp['edge']['b1']), w2=lp['edge']['W2'],
                       b2=r1(lp['edge']['b2']), g=r1(lp['edge']['g']),
                       be=r1(lp['edge']['be'])))
    nw_ = []
    for lp in procs:
        w1 = lp['node']['W1']
        nw_.append(dict(v1n=w1[:H], v1a=w1[H:], b1=r1(lp['node']['b1']),
                        v2=lp['node']['W2'], b2=r1(lp['node']['b2']),
                        g=r1(lp['node']['g']), be=r1(lp['node']['be'])))

    f32 = jnp.float32

    enc_n_call = pl.pallas_call(
        _enc_node_body,
        grid=(grid_n,),
        in_specs=[
            pl.BlockSpec((BN, H), lambda i: (i, 0)),
            pl.BlockSpec((BN, d_out), lambda i: (i, 0)),
            _wspec(H, H), _wspec(d_out, H), _wspec(1, H),
            _wspec(H, H), _wspec(1, H), _wspec(1, H), _wspec(1, H),
            _wspec(H, H), _wspec(H, H), _wspec(1, H),
        ],
        out_specs=[
            pl.BlockSpec((BN, H), lambda i: (i, 0)),
            pl.BlockSpec((2, BN, H), lambda i: (0, i, 0)),
        ],
        out_shape=[
            jax.ShapeDtypeStruct((n, H), f32),
            jax.ShapeDtypeStruct((2, n, H), f32),
        ],
    )
    w1n = p_enc_n['W1']
    hn, tbl = enc_n_call(x, estimate, w1n[:H], w1n[H:], r1(p_enc_n['b1']),
                         p_enc_n['W2'], r1(p_enc_n['b2']), r1(p_enc_n['g']),
                         r1(p_enc_n['be']), ew[0]['w1s'], ew[0]['w1d'],
                         ew[0]['b1'])

    enc_e_call = pl.pallas_call(
        _enc_edge_body,
        grid=(grid_e,),
        in_specs=[
            pl.BlockSpec((BE, d_edge), lambda i: (i, 0)),
            _wspec(d_edge, H), _wspec(1, H), _wspec(H, H),
            _wspec(1, H), _wspec(1, H), _wspec(1, H),
        ],
        out_specs=pl.BlockSpec((BE, H), lambda i: (i, 0)),
        out_shape=jax.ShapeDtypeStruct((e, H), f32),
    )
    he = enc_e_call(edge_attr, p_enc_e['W1'], r1(p_enc_e['b1']),
                    p_enc_e['W2'], r1(p_enc_e['b2']), r1(p_enc_e['g']),
                    r1(p_enc_e['be']))

    deg_call = pl.kernel(
        functools.partial(_sc_deg_body, rows_per_tile, nchunk),
        out_type=jax.ShapeDtypeStruct((NC, n_pad, DW), f32),
        mesh=_sc_mesh(),
        scratch_types=[
            pltpu.VMEM((CH,), jnp.int32),
            pltpu.VMEM((CH, DW), f32),
            pltpu.VMEM_SHARED((n_pad, DW), f32),
            pltpu.SemaphoreType.DMA,
        ],
    )
    degp = deg_call(dst, jnp.ones((CH, DW), f32),
                    jnp.zeros((rows_per_tile, DW), f32))

    gather_call = pl.kernel(
        functools.partial(_sc_gather_body, nchunk),
        out_type=(jax.ShapeDtypeStruct((e, H), f32),
                  jax.ShapeDtypeStruct((e, H), f32)),
        mesh=_sc_mesh(),
        scratch_types=[
            pltpu.VMEM((K, CH), jnp.int32),
            pltpu.VMEM((K, CH), jnp.int32),
            pltpu.VMEM((K, CH, H), f32),
            pltpu.VMEM((K, CH, H), f32),
            pltpu.SemaphoreType.DMA,
            pltpu.SemaphoreType.DMA,
            pltpu.SemaphoreType.DMA,
        ],
    )

    scatter_call = pl.kernel(
        functools.partial(_sc_scatter_body, rows_per_tile, nchunk),
        out_type=jax.ShapeDtypeStruct((NC, n_pad, H), f32),
        mesh=_sc_mesh(),
        scratch_types=[
            pltpu.VMEM((K, CH), jnp.int32),
            pltpu.VMEM((K, CH, H), f32),
            pltpu.VMEM_SHARED((n_pad, H), f32),
            pltpu.SemaphoreType.DMA,
        ],
    )
    zeros_nh = jnp.zeros((rows_per_tile, H), f32)

    edge_call = pl.pallas_call(
        _edge_body,
        grid=(grid_e,),
        in_specs=[
            pl.BlockSpec((BE, H), lambda i: (i, 0)),
            pl.BlockSpec((BE, H), lambda i: (i, 0)),
            pl.BlockSpec((BE, H), lambda i: (i, 0)),
            _wspec(H, H), _wspec(H, H), _wspec(1, H),
            _wspec(1, H), _wspec(1, H),
        ],
        out_specs=pl.BlockSpec((BE, H), lambda i: (i, 0)),
        out_shape=jax.ShapeDtypeStruct((e, H), f32),
    )
    node_call = pl.pallas_call(
        _node_body,
        grid=(grid_n,),
        in_specs=[
            pl.BlockSpec((BN, H), lambda i: (i, 0)),
            pl.BlockSpec((2, BN, H), lambda i: (0, i, 0)),
            pl.BlockSpec((2, BN, DW), lambda i: (0, i, 0)),
            _wspec(H, H), _wspec(H, H), _wspec(1, H),
            _wspec(H, H), _wspec(1, H), _wspec(1, H), _wspec(1, H),
            _wspec(H, H), _wspec(H, H), _wspec(1, H),
        ],
        out_specs=[
            pl.BlockSpec((BN, H), lambda i: (i, 0)),
            pl.BlockSpec((2, BN, H), lambda i: (0, i, 0)),
        ],
        out_shape=[
            jax.ShapeDtypeStruct((n, H), f32),
            jax.ShapeDtypeStruct((2, n, H), f32),
        ],
    )

    zero_w = jnp.zeros((H, H), f32)
    zero_b = jnp.zeros((1, H), f32)
    for l in range(nlayers):
        g1, g2 = gather_call(tbl.reshape(2 * n, H), src, dstn)
        ewl = ew[l]
        he = edge_call(he, g1, g2, ewl['w1e'], ewl['w2'], ewl['b2'],
                       ewl['g'], ewl['be'])
        s = scatter_call(he, dst, zeros_nh)
        nwl = nw_[l]
        if l + 1 < nlayers:
            w1s_n, w1d_n, b1_n = (ew[l + 1]['w1s'], ew[l + 1]['w1d'],
                                  ew[l + 1]['b1'])
        else:
            w1s_n, w1d_n, b1_n = zero_w, zero_w, zero_b
        hn, tbl = node_call(hn, s, degp, nwl['v1n'], nwl['v1a'], nwl['b1'],
                            nwl['v2'], nwl['b2'], nwl['g'], nwl['be'],
                            w1s_n, w1d_n, b1_n)

    dec_call = pl.pallas_call(
        _dec_body,
        grid=(grid_n,),
        in_specs=[
            pl.BlockSpec((BN, H), lambda i: (i, 0)),
            pl.BlockSpec((BN, d_out), lambda i: (i, 0)),
            _wspec(H, H), _wspec(1, H), _wspec(H, d_out), _wspec(1, d_out),
        ],
        out_specs=pl.BlockSpec((BN, d_out), lambda i: (i, 0)),
        out_shape=jax.ShapeDtypeStruct((n, d_out), f32),
    )
    out = dec_call(hn, estimate, p_dec['W1'], r1(p_dec['b1']),
                   p_dec['W2'], r1(p_dec['b2']))
    return out


# speedup vs baseline: 4.4169x; 1.0679x over previous
import functools

import jax
import jax.numpy as jnp
from jax import lax
from jax.experimental import pallas as pl
from jax.experimental.pallas import tpu as pltpu
from jax.experimental.pallas import tpu_sc as plsc

H = 128
NC, NS = 2, 16
NW = NC * NS
CH = 128
DW = 128

BN = 2000
BE = 4000


def _ln(h, g, be):
    mu = jnp.mean(h, axis=1, keepdims=True)
    var = jnp.mean((h - mu) ** 2, axis=1, keepdims=True)
    return (h - mu) / jnp.sqrt(var + 1e-5) * g + be


def _dot(a, b):
    return jnp.dot(a, b, preferred_element_type=jnp.float32)


def _enc_node_body(x_ref, est_ref, w1x, w1e, b1, w2, b2, g, be,
                   w1s, w1d, b1e, hn_ref, t_ref):
    t = _dot(x_ref[...], w1x[...]) + _dot(est_ref[...], w1e[...]) + b1[...]
    t = jnp.maximum(t, 0.0)
    h = _dot(t, w2[...]) + b2[...]
    hn = _ln(h, g[...], be[...])
    hn_ref[...] = hn
    t_ref[0] = _dot(hn, w1s[...])
    t_ref[1] = _dot(hn, w1d[...]) + b1e[...]


def _enc_edge_body(ea_ref, w1, b1, w2, b2, g, be, he_ref):
    t = _dot(ea_ref[...], w1[...]) + b1[...]
    t = jnp.maximum(t, 0.0)
    h = _dot(t, w2[...]) + b2[...]
    he_ref[...] = _ln(h, g[...], be[...])


def _edge_body(he_ref, g1_ref, w1e, w2, b2, g, be, out_ref):
    t = _dot(he_ref[...], w1e[...]) + g1_ref[...]
    t = jnp.maximum(t, 0.0)
    h = _dot(t, w2[...]) + b2[...]
    out_ref[...] = he_ref[...] + _ln(h, g[...], be[...])


def _node_body(hn_ref, s_ref, d_ref, v1n, v1a, b1, v2, b2, g, be,
               w1s, w1d, b1e, hn_out, t_ref):
    deg = d_ref[0][:, :1] + d_ref[1][:, :1]
    deg = jnp.maximum(deg, 1.0)
    agg = (s_ref[0] + s_ref[1]) / deg
    t = _dot(hn_ref[...], v1n[...]) + _dot(agg, v1a[...]) + b1[...]
    t = jnp.maximum(t, 0.0)
    h = _dot(t, v2[...]) + b2[...]
    hn = hn_ref[...] + _ln(h, g[...], be[...])
    hn_out[...] = hn
    t_ref[0] = _dot(hn, w1s[...])
    t_ref[1] = _dot(hn, w1d[...]) + b1e[...]


def _dec_body(hn_ref, est_ref, w1, b1, w2, b2, out_ref):
    t = _dot(hn_ref[...], w1[...]) + b1[...]
    t = jnp.maximum(t, 0.0)
    out_ref[...] = _dot(t, w2[...]) + b2[...] + est_ref[...]


def _wspec(r, c):
    return pl.BlockSpec((r, c), lambda i: (0, 0))


def _sc_mesh():
    return plsc.VectorSubcoreMesh(core_axis_name="c", subcore_axis_name="s",
                                  num_cores=NC, num_subcores=NS)


def _chunk_counts(wid, nchunk):
    base = nchunk // NW
    rem = nchunk - base * NW
    return base + jnp.where(wid < rem, 1, 0)


K = 3


def _sc_gather_body(nchunk, t_ref, src_ref, dstn_ref, g1_ref,
                    idx_a, idx_b, rows, sem_i, sem_g, sem_w):
    wid = lax.axis_index("s") * NC + lax.axis_index("c")
    n_mine = _chunk_counts(wid, nchunk)
    n_full = n_mine // K

    def group(j, carry):
        offs = [(wid + (K * j + k) * NW) * CH for k in range(K)]
        cps = []
        for k in range(K):
            cps.append(pltpu.async_copy(
                src_ref.at[pl.ds(offs[k], CH)], idx_a.at[k], sem_i))
            cps.append(pltpu.async_copy(
                dstn_ref.at[pl.ds(offs[k], CH)], idx_b.at[k], sem_i))
        for cp in cps:
            cp.wait()
        cps = []
        for k in range(K):
            cps.append(pltpu.async_copy(
                t_ref.at[idx_a.at[k]], rows.at[k], sem_g))
        for cp in cps:
            cp.wait()
        cps = []
        for k in range(K):
            cps.append(pltpu.async_copy(
                t_ref.at[idx_b.at[k]], rows.at[k], sem_g, add=True))
        for cp in cps:
            cp.wait()
        cps = []
        for k in range(K):
            cps.append(pltpu.async_copy(
                rows.at[k], g1_ref.at[pl.ds(offs[k], CH)], sem_w))
        for cp in cps:
            cp.wait()
        return carry

    lax.fori_loop(0, n_full, group, 0)

    def tail(i, carry):
        off = (wid + i * NW) * CH
        pltpu.sync_copy(src_ref.at[pl.ds(off, CH)], idx_a.at[0])
        pltpu.sync_copy(dstn_ref.at[pl.ds(off, CH)], idx_b.at[0])
        pltpu.async_copy(t_ref.at[idx_a.at[0]], rows.at[0], sem_g).wait()
        pltpu.async_copy(t_ref.at[idx_b.at[0]], rows.at[0], sem_g,
                         add=True).wait()
        pltpu.sync_copy(rows.at[0], g1_ref.at[pl.ds(off, CH)])
        return carry

    lax.fori_loop(K * n_full, n_mine, tail, 0)


def _sc_scatter_body(rows_per_tile, nchunk, he_ref, dst_ref, zeros_ref,
                     out_ref, idx_v, rows_v, acc, sem):
    cid = lax.axis_index("c")
    sid = lax.axis_index("s")
    wid = sid * NC + cid
    r0 = sid * rows_per_tile
    pltpu.sync_copy(zeros_ref, acc.at[pl.ds(r0, rows_per_tile)])
    plsc.subcore_barrier()
    n_mine = _chunk_counts(wid, nchunk)
    n_full = n_mine // K

    def group(j, carry):
        offs = [(wid + (K * j + k) * NW) * CH for k in range(K)]
        cps = []
        for k in range(K):
            cps.append(pltpu.async_copy(
                dst_ref.at[pl.ds(offs[k], CH)], idx_v.at[k], sem))
            cps.append(pltpu.async_copy(
                he_ref.at[pl.ds(offs[k], CH)], rows_v.at[k], sem))
        for cp in cps:
            cp.wait()
        cps = []
        for k in range(K):
            cps.append(pltpu.async_copy(
                rows_v.at[k], acc.at[idx_v.at[k]], sem, add=True))
        for cp in cps:
            cp.wait()
        return carry

    lax.fori_loop(0, n_full, group, 0)

    def tail(i, carry):
        off = (wid + i * NW) * CH
        pltpu.sync_copy(dst_ref.at[pl.ds(off, CH)], idx_v.at[0])
        pltpu.sync_copy(he_ref.at[pl.ds(off, CH)], rows_v.at[0])
        pltpu.sync_copy(rows_v.at[0], acc.at[idx_v.at[0]], add=True)
        return carry

    lax.fori_loop(K * n_full, n_mine, tail, 0)
    plsc.subcore_barrier()
    pltpu.sync_copy(acc.at[pl.ds(r0, rows_per_tile)],
                    out_ref.at[cid, pl.ds(r0, rows_per_tile)])


def kernel(x, edge_index, edge_attr, estimate, params):
    n = x.shape[0]
    e = edge_index.shape[1]
    d_out = estimate.shape[1]
    d_edge = edge_attr.shape[1]
    assert e % CH == 0 and e % BE == 0 and n % BN == 0
    nchunk = e // CH
    rows_per_tile = (n + NS * 8 - 1) // (NS * 8) * 8
    n_pad = rows_per_tile * NS
    grid_n = n // BN
    grid_e = e // BE

    src = edge_index[0]
    dst = edge_index[1]
    dstn = dst + n

    p_enc_n = params['enc_n']
    p_enc_e = params['enc_e']
    p_dec = params['dec']
    procs = params['proc']
    nlayers = len(procs)

    def r1(v):
        return v.reshape(1, -1)

    ew = []
    for lp in procs:
        w1 = lp['edge']['W1']
        ew.append(dict(w1e=w1[:H], w1s=w1[H:2 * H], w1d=w1[2 * H:],
                       b1=r1(lp['edge']['b1']), w2=lp['edge']['W2'],
                       b2=r1(lp['edge']['b2']), g=r1(lp['edge']['g']),
                       be=r1(lp['edge']['be'])))
    nw_ = []
    for lp in procs:
        w1 = lp['node']['W1']
        nw_.append(dict(v1n=w1[:H], v1a=w1[H:], b1=r1(lp['node']['b1']),
                        v2=lp['node']['W2'], b2=r1(lp['node']['b2']),
                        g=r1(lp['node']['g']), be=r1(lp['node']['be'])))

    f32 = jnp.float32

    enc_n_call = pl.pallas_call(
        _enc_node_body,
        grid=(grid_n,),
        in_specs=[
            pl.BlockSpec((BN, H), lambda i: (i, 0)),
            pl.BlockSpec((BN, d_out), lambda i: (i, 0)),
            _wspec(H, H), _wspec(d_out, H), _wspec(1, H),
            _wspec(H, H), _wspec(1, H), _wspec(1, H), _wspec(1, H),
            _wspec(H, H), _wspec(H, H), _wspec(1, H),
        ],
        out_specs=[
            pl.BlockSpec((BN, H), lambda i: (i, 0)),
            pl.BlockSpec((2, BN, H), lambda i: (0, i, 0)),
        ],
        out_shape=[
            jax.ShapeDtypeStruct((n, H), f32),
            jax.ShapeDtypeStruct((2, n, H), f32),
        ],
    )
    w1n = p_enc_n['W1']
    hn, tbl = enc_n_call(x, estimate, w1n[:H], w1n[H:], r1(p_enc_n['b1']),
                         p_enc_n['W2'], r1(p_enc_n['b2']), r1(p_enc_n['g']),
                         r1(p_enc_n['be']), ew[0]['w1s'], ew[0]['w1d'],
                         ew[0]['b1'])

    enc_e_call = pl.pallas_call(
        _enc_edge_body,
        grid=(grid_e,),
        in_specs=[
            pl.BlockSpec((BE, d_edge), lambda i: (i, 0)),
            _wspec(d_edge, H), _wspec(1, H), _wspec(H, H),
            _wspec(1, H), _wspec(1, H), _wspec(1, H),
        ],
        out_specs=pl.BlockSpec((BE, H), lambda i: (i, 0)),
        out_shape=jax.ShapeDtypeStruct((e, H), f32),
    )
    he = enc_e_call(edge_attr, p_enc_e['W1'], r1(p_enc_e['b1']),
                    p_enc_e['W2'], r1(p_enc_e['b2']), r1(p_enc_e['g']),
                    r1(p_enc_e['be']))

    gather_call = pl.kernel(
        functools.partial(_sc_gather_body, nchunk),
        out_type=jax.ShapeDtypeStruct((e, H), f32),
        mesh=_sc_mesh(),
        scratch_types=[
            pltpu.VMEM((K, CH), jnp.int32),
            pltpu.VMEM((K, CH), jnp.int32),
            pltpu.VMEM((K, CH, H), f32),
            pltpu.SemaphoreType.DMA,
            pltpu.SemaphoreType.DMA,
            pltpu.SemaphoreType.DMA,
        ],
    )

    scatter_call = pl.kernel(
        functools.partial(_sc_scatter_body, rows_per_tile, nchunk),
        out_type=jax.ShapeDtypeStruct((NC, n_pad, H), f32),
        mesh=_sc_mesh(),
        scratch_types=[
            pltpu.VMEM((K, CH), jnp.int32),
            pltpu.VMEM((K, CH, H), f32),
            pltpu.VMEM_SHARED((n_pad, H), f32),
            pltpu.SemaphoreType.DMA,
        ],
    )
    zeros_nh = jnp.zeros((rows_per_tile, H), f32)

    degp = scatter_call(jnp.ones((e, H), f32), dst, zeros_nh)
    degp1 = degp[:, :, :1]

    edge_call = pl.pallas_call(
        _edge_body,
        grid=(grid_e,),
        in_specs=[
            pl.BlockSpec((BE, H), lambda i: (i, 0)),
            pl.BlockSpec((BE, H), lambda i: (i, 0)),
            _wspec(H, H), _wspec(H, H), _wspec(1, H),
            _wspec(1, H), _wspec(1, H),
        ],
        out_specs=pl.BlockSpec((BE, H), lambda i: (i, 0)),
        out_shape=jax.ShapeDtypeStruct((e, H), f32),
    )
    node_call = pl.pallas_call(
        _node_body,
        grid=(grid_n,),
        in_specs=[
            pl.BlockSpec((BN, H), lambda i: (i, 0)),
            pl.BlockSpec((2, BN, H), lambda i: (0, i, 0)),
            pl.BlockSpec((2, BN, 1), lambda i: (0, i, 0)),
            _wspec(H, H), _wspec(H, H), _wspec(1, H),
            _wspec(H, H), _wspec(1, H), _wspec(1, H), _wspec(1, H),
            _wspec(H, H), _wspec(H, H), _wspec(1, H),
        ],
        out_specs=[
            pl.BlockSpec((BN, H), lambda i: (i, 0)),
            pl.BlockSpec((2, BN, H), lambda i: (0, i, 0)),
        ],
        out_shape=[
            jax.ShapeDtypeStruct((n, H), f32),
            jax.ShapeDtypeStruct((2, n, H), f32),
        ],
    )

    zero_w = jnp.zeros((H, H), f32)
    zero_b = jnp.zeros((1, H), f32)
    for l in range(nlayers):
        g1 = gather_call(tbl.reshape(2 * n, H), src, dstn)
        ewl = ew[l]
        he = edge_call(he, g1, ewl['w1e'], ewl['w2'], ewl['b2'],
                       ewl['g'], ewl['be'])
        s = scatter_call(he, dst, zeros_nh)
        nwl = nw_[l]
        if l + 1 < nlayers:
            w1s_n, w1d_n, b1_n = (ew[l + 1]['w1s'], ew[l + 1]['w1d'],
                                  ew[l + 1]['b1'])
        else:
            w1s_n, w1d_n, b1_n = zero_w, zero_w, zero_b
        hn, tbl = node_call(hn, s, degp1, nwl['v1n'], nwl['v1a'], nwl['b1'],
                            nwl['v2'], nwl['b2'], nwl['g'], nwl['be'],
                            w1s_n, w1d_n, b1_n)

    dec_call = pl.pallas_call(
        _dec_body,
        grid=(grid_n,),
        in_specs=[
            pl.BlockSpec((BN, H), lambda i: (i, 0)),
            pl.BlockSpec((BN, d_out), lambda i: (i, 0)),
            _wspec(H, H), _wspec(1, H), _wspec(H, d_out), _wspec(1, d_out),
        ],
        out_specs=pl.BlockSpec((BN, d_out), lambda i: (i, 0)),
        out_shape=jax.ShapeDtypeStruct((n, d_out), f32),
    )
    out = dec_call(hn, estimate, p_dec['W1'], r1(p_dec['b1']),
                   p_dec['W2'], r1(p_dec['b2']))
    return out


# speedup vs baseline: 4.4250x; 1.0018x over previous
import functools

import jax
import jax.numpy as jnp
from jax import lax
from jax.experimental import pallas as pl
from jax.experimental.pallas import tpu as pltpu
from jax.experimental.pallas import tpu_sc as plsc

H = 128
NC, NS = 2, 16
NW = NC * NS
CH = 128
DW = 128

BN = 2000
BE = 4000


def _ln(h, g, be):
    mu = jnp.mean(h, axis=1, keepdims=True)
    var = jnp.mean((h - mu) ** 2, axis=1, keepdims=True)
    return (h - mu) / jnp.sqrt(var + 1e-5) * g + be


def _dot(a, b):
    return jnp.dot(a, b, preferred_element_type=jnp.float32)


def _enc_node_body(x_ref, est_ref, w1x, w1e, b1, w2, b2, g, be,
                   w1s, w1d, b1e, hn_ref, t_ref):
    t = _dot(x_ref[...], w1x[...]) + _dot(est_ref[...], w1e[...]) + b1[...]
    t = jnp.maximum(t, 0.0)
    h = _dot(t, w2[...]) + b2[...]
    hn = _ln(h, g[...], be[...])
    hn_ref[...] = hn
    t_ref[0] = _dot(hn, w1s[...])
    t_ref[1] = _dot(hn, w1d[...]) + b1e[...]


def _enc_edge_body(ea_ref, w1, b1, w2, b2, g, be, he_ref):
    t = _dot(ea_ref[...], w1[...]) + b1[...]
    t = jnp.maximum(t, 0.0)
    h = _dot(t, w2[...]) + b2[...]
    he_ref[...] = _ln(h, g[...], be[...])


def _edge_body(he_ref, g1_ref, w1e, w2, b2, g, be, out_ref):
    t = _dot(he_ref[...], w1e[...]) + g1_ref[...]
    t = jnp.maximum(t, 0.0)
    h = _dot(t, w2[...]) + b2[...]
    out_ref[...] = he_ref[...] + _ln(h, g[...], be[...])


def _node_body(hn_ref, s_ref, d_ref, v1n, v1a, b1, v2, b2, g, be,
               w1s, w1d, b1e, hn_out, t_ref):
    deg = d_ref[0][:, :1] + d_ref[1][:, :1]
    deg = jnp.maximum(deg, 1.0)
    agg = (s_ref[0] + s_ref[1]) / deg
    t = _dot(hn_ref[...], v1n[...]) + _dot(agg, v1a[...]) + b1[...]
    t = jnp.maximum(t, 0.0)
    h = _dot(t, v2[...]) + b2[...]
    hn = hn_ref[...] + _ln(h, g[...], be[...])
    hn_out[...] = hn
    t_ref[0] = _dot(hn, w1s[...])
    t_ref[1] = _dot(hn, w1d[...]) + b1e[...]


def _dec_body(hn_ref, est_ref, w1, b1, w2, b2, out_ref):
    t = _dot(hn_ref[...], w1[...]) + b1[...]
    t = jnp.maximum(t, 0.0)
    out_ref[...] = _dot(t, w2[...]) + b2[...] + est_ref[...]


def _wspec(r, c):
    return pl.BlockSpec((r, c), lambda i: (0, 0))


def _sc_mesh():
    return plsc.VectorSubcoreMesh(core_axis_name="c", subcore_axis_name="s",
                                  num_cores=NC, num_subcores=NS)


def _chunk_range(wid, nchunk):
    start = wid * nchunk // NW
    end = (wid + 1) * nchunk // NW
    return start, end


K = 3


def _sc_gather_body(nchunk, t_ref, src_ref, dstn_ref, g1_ref,
                    idx_a, idx_b, rows, sem_i, sem_g, sem_w):
    wid = lax.axis_index("s") * NC + lax.axis_index("c")
    start, end = _chunk_range(wid, nchunk)
    n_mine = end - start
    n_full = n_mine // K

    def group(j, carry):
        c0 = start + K * j
        off = c0 * CH
        cps = []
        for k in range(K):
            cps.append(pltpu.async_copy(
                src_ref.at[pl.ds(off + k * CH, CH)], idx_a.at[k], sem_i))
            cps.append(pltpu.async_copy(
                dstn_ref.at[pl.ds(off + k * CH, CH)], idx_b.at[k], sem_i))
        for cp in cps:
            cp.wait()
        cps = [pltpu.async_copy(t_ref.at[idx_a.at[k]],
                                rows.at[pl.ds(k * CH, CH)], sem_g)
               for k in range(K)]
        for cp in cps:
            cp.wait()
        cps = [pltpu.async_copy(t_ref.at[idx_b.at[k]],
                                rows.at[pl.ds(k * CH, CH)], sem_g, add=True)
               for k in range(K)]
        for cp in cps:
            cp.wait()
        pltpu.async_copy(rows, g1_ref.at[pl.ds(off, K * CH)], sem_w).wait()
        return carry

    lax.fori_loop(0, n_full, group, 0)

    def tail(i, carry):
        off = (start + i) * CH
        pltpu.sync_copy(src_ref.at[pl.ds(off, CH)], idx_a.at[0])
        pltpu.sync_copy(dstn_ref.at[pl.ds(off, CH)], idx_b.at[0])
        pltpu.async_copy(t_ref.at[idx_a.at[0]],
                         rows.at[pl.ds(0, CH)], sem_g).wait()
        pltpu.async_copy(t_ref.at[idx_b.at[0]],
                         rows.at[pl.ds(0, CH)], sem_g, add=True).wait()
        pltpu.sync_copy(rows.at[pl.ds(0, CH)], g1_ref.at[pl.ds(off, CH)])
        return carry

    lax.fori_loop(K * n_full, n_mine, tail, 0)


def _sc_scatter_body(rows_per_tile, nchunk, he_ref, dst_ref, zeros_ref,
                     out_ref, idx_v, rows_v, acc, sem):
    cid = lax.axis_index("c")
    sid = lax.axis_index("s")
    wid = sid * NC + cid
    r0 = sid * rows_per_tile
    pltpu.sync_copy(zeros_ref, acc.at[pl.ds(r0, rows_per_tile)])
    plsc.subcore_barrier()
    start, end = _chunk_range(wid, nchunk)
    n_mine = end - start
    n_full = n_mine // K

    def group(j, carry):
        c0 = start + K * j
        off = c0 * CH
        cps = [pltpu.async_copy(he_ref.at[pl.ds(off, K * CH)], rows_v, sem)]
        for k in range(K):
            cps.append(pltpu.async_copy(
                dst_ref.at[pl.ds(off + k * CH, CH)], idx_v.at[k], sem))
        for cp in cps:
            cp.wait()
        cps = [pltpu.async_copy(rows_v.at[pl.ds(k * CH, CH)],
                                acc.at[idx_v.at[k]], sem, add=True)
               for k in range(K)]
        for cp in cps:
            cp.wait()
        return carry

    lax.fori_loop(0, n_full, group, 0)

    def tail(i, carry):
        off = (start + i) * CH
        pltpu.sync_copy(dst_ref.at[pl.ds(off, CH)], idx_v.at[0])
        pltpu.sync_copy(he_ref.at[pl.ds(off, CH)], rows_v.at[pl.ds(0, CH)])
        pltpu.sync_copy(rows_v.at[pl.ds(0, CH)], acc.at[idx_v.at[0]],
                        add=True)
        return carry

    lax.fori_loop(K * n_full, n_mine, tail, 0)
    plsc.subcore_barrier()
    pltpu.sync_copy(acc.at[pl.ds(r0, rows_per_tile)],
                    out_ref.at[cid, pl.ds(r0, rows_per_tile)])


def kernel(x, edge_index, edge_attr, estimate, params):
    n = x.shape[0]
    e = edge_index.shape[1]
    d_out = estimate.shape[1]
    d_edge = edge_attr.shape[1]
    assert e % CH == 0 and e % BE == 0 and n % BN == 0
    nchunk = e // CH
    rows_per_tile = (n + NS * 8 - 1) // (NS * 8) * 8
    n_pad = rows_per_tile * NS
    grid_n = n // BN
    grid_e = e // BE

    src = edge_index[0]
    dst = edge_index[1]
    dstn = dst + n

    p_enc_n = params['enc_n']
    p_enc_e = params['enc_e']
    p_dec = params['dec']
    procs = params['proc']
    nlayers = len(procs)

    def r1(v):
        return v.reshape(1, -1)

    ew = []
    for lp in procs:
        w1 = lp['edge']['W1']
        ew.append(dict(w1e=w1[:H], w1s=w1[H:2 * H], w1d=w1[2 * H:],
                       b1=r1(lp['edge']['b1']), w2=lp['edge']['W2'],
                       b2=r1(lp['edge']['b2']), g=r1(lp['edge']['g']),
                       be=r1(lp['edge']['be'])))
    nw_ = []
    for lp in procs:
        w1 = lp['node']['W1']
        nw_.append(dict(v1n=w1[:H], v1a=w1[H:], b1=r1(lp['node']['b1']),
                        v2=lp['node']['W2'], b2=r1(lp['node']['b2']),
                        g=r1(lp['node']['g']), be=r1(lp['node']['be'])))

    f32 = jnp.float32

    enc_n_call = pl.pallas_call(
        _enc_node_body,
        grid=(grid_n,),
        in_specs=[
            pl.BlockSpec((BN, H), lambda i: (i, 0)),
            pl.BlockSpec((BN, d_out), lambda i: (i, 0)),
            _wspec(H, H), _wspec(d_out, H), _wspec(1, H),
            _wspec(H, H), _wspec(1, H), _wspec(1, H), _wspec(1, H),
            _wspec(H, H), _wspec(H, H), _wspec(1, H),
        ],
        out_specs=[
            pl.BlockSpec((BN, H), lambda i: (i, 0)),
            pl.BlockSpec((2, BN, H), lambda i: (0, i, 0)),
        ],
        out_shape=[
            jax.ShapeDtypeStruct((n, H), f32),
            jax.ShapeDtypeStruct((2, n, H), f32),
        ],
    )
    w1n = p_enc_n['W1']
    hn, tbl = enc_n_call(x, estimate, w1n[:H], w1n[H:], r1(p_enc_n['b1']),
                         p_enc_n['W2'], r1(p_enc_n['b2']), r1(p_enc_n['g']),
                         r1(p_enc_n['be']), ew[0]['w1s'], ew[0]['w1d'],
                         ew[0]['b1'])

    enc_e_call = pl.pallas_call(
        _enc_edge_body,
        grid=(grid_e,),
        in_specs=[
            pl.BlockSpec((BE, d_edge), lambda i: (i, 0)),
            _wspec(d_edge, H), _wspec(1, H), _wspec(H, H),
            _wspec(1, H), _wspec(1, H), _wspec(1, H),
        ],
        out_specs=pl.BlockSpec((BE, H), lambda i: (i, 0)),
        out_shape=jax.ShapeDtypeStruct((e, H), f32),
    )
    he = enc_e_call(edge_attr, p_enc_e['W1'], r1(p_enc_e['b1']),
                    p_enc_e['W2'], r1(p_enc_e['b2']), r1(p_enc_e['g']),
                    r1(p_enc_e['be']))

    gather_call = pl.kernel(
        functools.partial(_sc_gather_body, nchunk),
        out_type=jax.ShapeDtypeStruct((e, H), f32),
        mesh=_sc_mesh(),
        scratch_types=[
            pltpu.VMEM((K, CH), jnp.int32),
            pltpu.VMEM((K, CH), jnp.int32),
            pltpu.VMEM((K * CH, H), f32),
            pltpu.SemaphoreType.DMA,
            pltpu.SemaphoreType.DMA,
            pltpu.SemaphoreType.DMA,
        ],
    )

    scatter_call = pl.kernel(
        functools.partial(_sc_scatter_body, rows_per_tile, nchunk),
        out_type=jax.ShapeDtypeStruct((NC, n_pad, H), f32),
        mesh=_sc_mesh(),
        scratch_types=[
            pltpu.VMEM((K, CH), jnp.int32),
            pltpu.VMEM((K * CH, H), f32),
            pltpu.VMEM_SHARED((n_pad, H), f32),
            pltpu.SemaphoreType.DMA,
        ],
    )
    zeros_nh = jnp.zeros((rows_per_tile, H), f32)

    degp = scatter_call(jnp.ones((e, H), f32), dst, zeros_nh)
    degp1 = degp[:, :, :1]

    edge_call = pl.pallas_call(
        _edge_body,
        grid=(grid_e,),
        in_specs=[
            pl.BlockSpec((BE, H), lambda i: (i, 0)),
            pl.BlockSpec((BE, H), lambda i: (i, 0)),
            _wspec(H, H), _wspec(H, H), _wspec(1, H),
            _wspec(1, H), _wspec(1, H),
        ],
        out_specs=pl.BlockSpec((BE, H), lambda i: (i, 0)),
        out_shape=jax.ShapeDtypeStruct((e, H), f32),
    )
    node_call = pl.pallas_call(
        _node_body,
        grid=(grid_n,),
        in_specs=[
            pl.BlockSpec((BN, H), lambda i: (i, 0)),
            pl.BlockSpec((2, BN, H), lambda i: (0, i, 0)),
            pl.BlockSpec((2, BN, 1), lambda i: (0, i, 0)),
            _wspec(H, H), _wspec(H, H), _wspec(1, H),
            _wspec(H, H), _wspec(1, H), _wspec(1, H), _wspec(1, H),
            _wspec(H, H), _wspec(H, H), _wspec(1, H),
        ],
        out_specs=[
            pl.BlockSpec((BN, H), lambda i: (i, 0)),
            pl.BlockSpec((2, BN, H), lambda i: (0, i, 0)),
        ],
        out_shape=[
            jax.ShapeDtypeStruct((n, H), f32),
            jax.ShapeDtypeStruct((2, n, H), f32),
        ],
    )

    zero_w = jnp.zeros((H, H), f32)
    zero_b = jnp.zeros((1, H), f32)
    for l in range(nlayers):
        g1 = gather_call(tbl.reshape(2 * n, H), src, dstn)
        ewl = ew[l]
        he = edge_call(he, g1, ewl['w1e'], ewl['w2'], ewl['b2'],
                       ewl['g'], ewl['be'])
        s = scatter_call(he, dst, zeros_nh)
        nwl = nw_[l]
        if l + 1 < nlayers:
            w1s_n, w1d_n, b1_n = (ew[l + 1]['w1s'], ew[l + 1]['w1d'],
                                  ew[l + 1]['b1'])
        else:
            w1s_n, w1d_n, b1_n = zero_w, zero_w, zero_b
        hn, tbl = node_call(hn, s, degp1, nwl['v1n'], nwl['v1a'], nwl['b1'],
                            nwl['v2'], nwl['b2'], nwl['g'], nwl['be'],
                            w1s_n, w1d_n, b1_n)

    dec_call = pl.pallas_call(
        _dec_body,
        grid=(grid_n,),
        in_specs=[
            pl.BlockSpec((BN, H), lambda i: (i, 0)),
            pl.BlockSpec((BN, d_out), lambda i: (i, 0)),
            _wspec(H, H), _wspec(1, H), _wspec(H, d_out), _wspec(1, d_out),
        ],
        out_specs=pl.BlockSpec((BN, d_out), lambda i: (i, 0)),
        out_shape=jax.ShapeDtypeStruct((n, d_out), f32),
    )
    out = dec_call(hn, estimate, p_dec['W1'], r1(p_dec['b1']),
                   p_dec['W2'], r1(p_dec['b2']))
    return out


# speedup vs baseline: 4.7224x; 1.0672x over previous
import functools

import jax
import jax.numpy as jnp
from jax import lax
from jax.experimental import pallas as pl
from jax.experimental.pallas import tpu as pltpu
from jax.experimental.pallas import tpu_sc as plsc

H = 128
NC, NS = 2, 16
NW = NC * NS
CH = 128
DW = 128

BN = 2000
BE = 4000


def _ln(h, g, be):
    mu = jnp.mean(h, axis=1, keepdims=True)
    var = jnp.mean((h - mu) ** 2, axis=1, keepdims=True)
    return (h - mu) / jnp.sqrt(var + 1e-5) * g + be


def _dot(a, b):
    return jnp.dot(a, b, preferred_element_type=jnp.float32)


def _enc_node_body(x_ref, est_ref, w1x, w1e, b1, w2, b2, g, be,
                   w1s, w1d, b1e, hn_ref, t_ref):
    t = _dot(x_ref[...], w1x[...]) + _dot(est_ref[...], w1e[...]) + b1[...]
    t = jnp.maximum(t, 0.0)
    h = _dot(t, w2[...]) + b2[...]
    hn = _ln(h, g[...], be[...])
    hn_ref[...] = hn
    t_ref[0] = _dot(hn, w1s[...])
    t_ref[1] = _dot(hn, w1d[...]) + b1e[...]


def _enc_edge_body(ea_ref, w1, b1, w2, b2, g, be, he_ref):
    t = _dot(ea_ref[...], w1[...]) + b1[...]
    t = jnp.maximum(t, 0.0)
    h = _dot(t, w2[...]) + b2[...]
    he_ref[...] = _ln(h, g[...], be[...])


def _edge_body(he_ref, g1_ref, w1e, w2, b2, g, be, out_ref):
    t = _dot(he_ref[...], w1e[...]) + g1_ref[...]
    t = jnp.maximum(t, 0.0)
    h = _dot(t, w2[...]) + b2[...]
    out_ref[...] = he_ref[...] + _ln(h, g[...], be[...])


def _node_body(hn_ref, s_ref, d_ref, v1n, v1a, b1, v2, b2, g, be,
               w1s, w1d, b1e, hn_out, t_ref):
    deg = d_ref[0][:, :1] + d_ref[1][:, :1]
    deg = jnp.maximum(deg, 1.0)
    agg = (s_ref[0] + s_ref[1]) / deg
    t = _dot(hn_ref[...], v1n[...]) + _dot(agg, v1a[...]) + b1[...]
    t = jnp.maximum(t, 0.0)
    h = _dot(t, v2[...]) + b2[...]
    hn = hn_ref[...] + _ln(h, g[...], be[...])
    hn_out[...] = hn
    t_ref[0] = _dot(hn, w1s[...])
    t_ref[1] = _dot(hn, w1d[...]) + b1e[...]


def _dec_body(hn_ref, est_ref, w1, b1, w2, b2, out_ref):
    t = _dot(hn_ref[...], w1[...]) + b1[...]
    t = jnp.maximum(t, 0.0)
    out_ref[...] = _dot(t, w2[...]) + b2[...] + est_ref[...]


def _wspec(r, c):
    return pl.BlockSpec((r, c), lambda i: (0, 0))


def _sc_mesh():
    return plsc.VectorSubcoreMesh(core_axis_name="c", subcore_axis_name="s",
                                  num_cores=NC, num_subcores=NS)


def _chunk_range(wid, nchunk, nworkers):
    start = wid * nchunk // nworkers
    end = (wid + 1) * nchunk // nworkers
    return start, end


K = 3


def _sc_gather_body(nchunk, t_ref, src_ref, dstn_ref, g1_ref,
                    idx_a, idx_b, rows, sem_i, sem_g, sem_w):
    wid = lax.axis_index("s") * NC + lax.axis_index("c")
    start, end = _chunk_range(wid, nchunk, NW)
    n_mine = end - start
    n_full = n_mine // K

    def group(j, carry):
        c0 = start + K * j
        off = c0 * CH
        cps = []
        for k in range(K):
            cps.append(pltpu.async_copy(
                src_ref.at[pl.ds(off + k * CH, CH)], idx_a.at[k], sem_i))
            cps.append(pltpu.async_copy(
                dstn_ref.at[pl.ds(off + k * CH, CH)], idx_b.at[k], sem_i))
        for cp in cps:
            cp.wait()
        cps = [pltpu.async_copy(t_ref.at[idx_a.at[k]],
                                rows.at[pl.ds(k * CH, CH)], sem_g)
               for k in range(K)]
        for cp in cps:
            cp.wait()
        cps = [pltpu.async_copy(t_ref.at[idx_b.at[k]],
                                rows.at[pl.ds(k * CH, CH)], sem_g, add=True)
               for k in range(K)]
        for cp in cps:
            cp.wait()
        pltpu.async_copy(rows, g1_ref.at[pl.ds(off, K * CH)], sem_w).wait()
        return carry

    lax.fori_loop(0, n_full, group, 0)

    def tail(i, carry):
        off = (start + i) * CH
        pltpu.sync_copy(src_ref.at[pl.ds(off, CH)], idx_a.at[0])
        pltpu.sync_copy(dstn_ref.at[pl.ds(off, CH)], idx_b.at[0])
        pltpu.async_copy(t_ref.at[idx_a.at[0]],
                         rows.at[pl.ds(0, CH)], sem_g).wait()
        pltpu.async_copy(t_ref.at[idx_b.at[0]],
                         rows.at[pl.ds(0, CH)], sem_g, add=True).wait()
        pltpu.sync_copy(rows.at[pl.ds(0, CH)], g1_ref.at[pl.ds(off, CH)])
        return carry

    lax.fori_loop(K * n_full, n_mine, tail, 0)


def _scatter_range(w, nchunk, he_ref, dst_ref, idx_v, rows_v, acc, sem):
    start, end = _chunk_range(w, nchunk, NW // 2)
    n_mine = end - start
    n_full = n_mine // K

    def group(j, carry):
        c0 = start + K * j
        off = c0 * CH
        cps = [pltpu.async_copy(he_ref.at[pl.ds(off, K * CH)], rows_v, sem)]
        for k in range(K):
            cps.append(pltpu.async_copy(
                dst_ref.at[pl.ds(off + k * CH, CH)], idx_v.at[k], sem))
        for cp in cps:
            cp.wait()
        cps = [pltpu.async_copy(rows_v.at[pl.ds(k * CH, CH)],
                                acc.at[idx_v.at[k]], sem, add=True)
               for k in range(K)]
        for cp in cps:
            cp.wait()
        return carry

    lax.fori_loop(0, n_full, group, 0)

    def tail(i, carry):
        off = (start + i) * CH
        pltpu.sync_copy(dst_ref.at[pl.ds(off, CH)], idx_v.at[0])
        pltpu.sync_copy(he_ref.at[pl.ds(off, CH)], rows_v.at[pl.ds(0, CH)])
        pltpu.sync_copy(rows_v.at[pl.ds(0, CH)], acc.at[idx_v.at[0]],
                        add=True)
        return carry

    lax.fori_loop(K * n_full, n_mine, tail, 0)


def _sc_scatter_body(rows_per_tile, nchunk_h, he_a, he_b, dst_a, dst_b,
                     zeros_ref, out_ref, idx_v, rows_v, acc, sem):
    cid = lax.axis_index("c")
    sid = lax.axis_index("s")
    wid = sid * NC + cid
    half = NW // 2
    r0 = sid * rows_per_tile
    pltpu.sync_copy(zeros_ref, acc.at[pl.ds(r0, rows_per_tile)])
    plsc.subcore_barrier()

    @pl.when(wid < half)
    def _():
        _scatter_range(wid, nchunk_h, he_a, dst_a, idx_v, rows_v, acc, sem)

    @pl.when(wid >= half)
    def _():
        _scatter_range(wid - half, nchunk_h, he_b, dst_b, idx_v, rows_v,
                       acc, sem)

    plsc.subcore_barrier()
    pltpu.sync_copy(acc.at[pl.ds(r0, rows_per_tile)],
                    out_ref.at[cid, pl.ds(r0, rows_per_tile)])


def kernel(x, edge_index, edge_attr, estimate, params):
    n = x.shape[0]
    e = edge_index.shape[1]
    d_out = estimate.shape[1]
    d_edge = edge_attr.shape[1]
    eh = e // 2
    assert eh % CH == 0 and eh % BE == 0 and n % BN == 0
    nchunk_h = eh // CH
    rows_per_tile = (n + NS * 8 - 1) // (NS * 8) * 8
    n_pad = rows_per_tile * NS
    grid_n = n // BN
    grid_e = eh // BE

    src = edge_index[0]
    dst = edge_index[1]
    dstn = dst + n
    src_h = (src[:eh], src[eh:])
    dst_h = (dst[:eh], dst[eh:])
    dstn_h = (dstn[:eh], dstn[eh:])

    p_enc_n = params['enc_n']
    p_enc_e = params['enc_e']
    p_dec = params['dec']
    procs = params['proc']
    nlayers = len(procs)

    def r1(v):
        return v.reshape(1, -1)

    ew = []
    for lp in procs:
        w1 = lp['edge']['W1']
        ew.append(dict(w1e=w1[:H], w1s=w1[H:2 * H], w1d=w1[2 * H:],
                       b1=r1(lp['edge']['b1']), w2=lp['edge']['W2'],
                       b2=r1(lp['edge']['b2']), g=r1(lp['edge']['g']),
                       be=r1(lp['edge']['be'])))
    nw_ = []
    for lp in procs:
        w1 = lp['node']['W1']
        nw_.append(dict(v1n=w1[:H], v1a=w1[H:], b1=r1(lp['node']['b1']),
                        v2=lp['node']['W2'], b2=r1(lp['node']['b2']),
                        g=r1(lp['node']['g']), be=r1(lp['node']['be'])))

    f32 = jnp.float32

    enc_n_call = pl.pallas_call(
        _enc_node_body,
        grid=(grid_n,),
        in_specs=[
            pl.BlockSpec((BN, H), lambda i: (i, 0)),
            pl.BlockSpec((BN, d_out), lambda i: (i, 0)),
            _wspec(H, H), _wspec(d_out, H), _wspec(1, H),
            _wspec(H, H), _wspec(1, H), _wspec(1, H), _wspec(1, H),
            _wspec(H, H), _wspec(H, H), _wspec(1, H),
        ],
        out_specs=[
            pl.BlockSpec((BN, H), lambda i: (i, 0)),
            pl.BlockSpec((2, BN, H), lambda i: (0, i, 0)),
        ],
        out_shape=[
            jax.ShapeDtypeStruct((n, H), f32),
            jax.ShapeDtypeStruct((2, n, H), f32),
        ],
    )
    w1n = p_enc_n['W1']
    hn, tbl = enc_n_call(x, estimate, w1n[:H], w1n[H:], r1(p_enc_n['b1']),
                         p_enc_n['W2'], r1(p_enc_n['b2']), r1(p_enc_n['g']),
                         r1(p_enc_n['be']), ew[0]['w1s'], ew[0]['w1d'],
                         ew[0]['b1'])

    enc_e_call = pl.pallas_call(
        _enc_edge_body,
        grid=(grid_e,),
        in_specs=[
            pl.BlockSpec((BE, d_edge), lambda i: (i, 0)),
            _wspec(d_edge, H), _wspec(1, H), _wspec(H, H),
            _wspec(1, H), _wspec(1, H), _wspec(1, H),
        ],
        out_specs=pl.BlockSpec((BE, H), lambda i: (i, 0)),
        out_shape=jax.ShapeDtypeStruct((eh, H), f32),
    )
    he_h = [enc_e_call(edge_attr[i * eh:(i + 1) * eh], p_enc_e['W1'],
                       r1(p_enc_e['b1']), p_enc_e['W2'], r1(p_enc_e['b2']),
                       r1(p_enc_e['g']), r1(p_enc_e['be']))
            for i in range(2)]

    gather_call = pl.kernel(
        functools.partial(_sc_gather_body, nchunk_h),
        out_type=jax.ShapeDtypeStruct((eh, H), f32),
        mesh=_sc_mesh(),
        scratch_types=[
            pltpu.VMEM((K, CH), jnp.int32),
            pltpu.VMEM((K, CH), jnp.int32),
            pltpu.VMEM((K * CH, H), f32),
            pltpu.SemaphoreType.DMA,
            pltpu.SemaphoreType.DMA,
            pltpu.SemaphoreType.DMA,
        ],
    )

    scatter_call = pl.kernel(
        functools.partial(_sc_scatter_body, rows_per_tile, nchunk_h),
        out_type=jax.ShapeDtypeStruct((NC, n_pad, H), f32),
        mesh=_sc_mesh(),
        scratch_types=[
            pltpu.VMEM((K, CH), jnp.int32),
            pltpu.VMEM((K * CH, H), f32),
            pltpu.VMEM_SHARED((n_pad, H), f32),
            pltpu.SemaphoreType.DMA,
        ],
    )
    zeros_nh = jnp.zeros((rows_per_tile, H), f32)

    ones_h = jnp.ones((eh, H), f32)
    degp = scatter_call(ones_h, ones_h, dst_h[0], dst_h[1], zeros_nh)
    degp1 = degp[:, :, :1]

    edge_call = pl.pallas_call(
        _edge_body,
        grid=(grid_e,),
        in_specs=[
            pl.BlockSpec((BE, H), lambda i: (i, 0)),
            pl.BlockSpec((BE, H), lambda i: (i, 0)),
            _wspec(H, H), _wspec(H, H), _wspec(1, H),
            _wspec(1, H), _wspec(1, H),
        ],
        out_specs=pl.BlockSpec((BE, H), lambda i: (i, 0)),
        out_shape=jax.ShapeDtypeStruct((eh, H), f32),
    )
    node_call = pl.pallas_call(
        _node_body,
        grid=(grid_n,),
        in_specs=[
            pl.BlockSpec((BN, H), lambda i: (i, 0)),
            pl.BlockSpec((2, BN, H), lambda i: (0, i, 0)),
            pl.BlockSpec((2, BN, 1), lambda i: (0, i, 0)),
            _wspec(H, H), _wspec(H, H), _wspec(1, H),
            _wspec(H, H), _wspec(1, H), _wspec(1, H), _wspec(1, H),
            _wspec(H, H), _wspec(H, H), _wspec(1, H),
        ],
        out_specs=[
            pl.BlockSpec((BN, H), lambda i: (i, 0)),
            pl.BlockSpec((2, BN, H), lambda i: (0, i, 0)),
        ],
        out_shape=[
            jax.ShapeDtypeStruct((n, H), f32),
            jax.ShapeDtypeStruct((2, n, H), f32),
        ],
    )

    zero_w = jnp.zeros((H, H), f32)
    zero_b = jnp.zeros((1, H), f32)
    for l in range(nlayers):
        tbl2 = tbl.reshape(2 * n, H)
        ewl = ew[l]
        for i in range(2):
            g1 = gather_call(tbl2, src_h[i], dstn_h[i])
            he_h[i] = edge_call(he_h[i], g1, ewl['w1e'], ewl['w2'],
                                ewl['b2'], ewl['g'], ewl['be'])
        s = scatter_call(he_h[0], he_h[1], dst_h[0], dst_h[1], zeros_nh)
        nwl = nw_[l]
        if l + 1 < nlayers:
            w1s_n, w1d_n, b1_n = (ew[l + 1]['w1s'], ew[l + 1]['w1d'],
                                  ew[l + 1]['b1'])
        else:
            w1s_n, w1d_n, b1_n = zero_w, zero_w, zero_b
        hn, tbl = node_call(hn, s, degp1, nwl['v1n'], nwl['v1a'], nwl['b1'],
                            nwl['v2'], nwl['b2'], nwl['g'], nwl['be'],
                            w1s_n, w1d_n, b1_n)

    dec_call = pl.pallas_call(
        _dec_body,
        grid=(grid_n,),
        in_specs=[
            pl.BlockSpec((BN, H), lambda i: (i, 0)),
            pl.BlockSpec((BN, d_out), lambda i: (i, 0)),
            _wspec(H, H), _wspec(1, H), _wspec(H, d_out), _wspec(1, d_out),
        ],
        out_specs=pl.BlockSpec((BN, d_out), lambda i: (i, 0)),
        out_shape=jax.ShapeDtypeStruct((n, d_out), f32),
    )
    out = dec_call(hn, estimate, p_dec['W1'], r1(p_dec['b1']),
                   p_dec['W2'], r1(p_dec['b2']))
    return out


# speedup vs baseline: 5.1134x; 1.0828x over previous
import functools

import jax
import jax.numpy as jnp
from jax import lax
from jax.experimental import pallas as pl
from jax.experimental.pallas import tpu as pltpu
from jax.experimental.pallas import tpu_sc as plsc

H = 128
NC, NS = 2, 16
NW = NC * NS
CH = 128
DW = 128

BN = 2000
BE = 4000


def _ln(h, g, be):
    mu = jnp.mean(h, axis=1, keepdims=True)
    var = jnp.mean((h - mu) ** 2, axis=1, keepdims=True)
    return (h - mu) / jnp.sqrt(var + 1e-5) * g + be


def _dot(a, b):
    return jnp.dot(a, b, preferred_element_type=jnp.float32)


def _enc_node_body(x_ref, est_ref, w1x, w1e, b1, w2, b2, g, be,
                   w1s, w1d, b1e, hn_ref, t_ref):
    t = _dot(x_ref[...], w1x[...]) + _dot(est_ref[...], w1e[...]) + b1[...]
    t = jnp.maximum(t, 0.0)
    h = _dot(t, w2[...]) + b2[...]
    hn = _ln(h, g[...], be[...])
    hn_ref[...] = hn
    t_ref[0] = _dot(hn, w1s[...])
    t_ref[1] = _dot(hn, w1d[...]) + b1e[...]


def _enc_edge_body(ea_ref, w1, b1, w2, b2, g, be, he_ref):
    t = _dot(ea_ref[...], w1[...]) + b1[...]
    t = jnp.maximum(t, 0.0)
    h = _dot(t, w2[...]) + b2[...]
    he_ref[...] = _ln(h, g[...], be[...])


def _edge_body(he_ref, g1_ref, w1e, w2, b2, g, be, out_ref):
    t = _dot(he_ref[...], w1e[...]) + g1_ref[...]
    t = jnp.maximum(t, 0.0)
    h = _dot(t, w2[...]) + b2[...]
    out_ref[...] = he_ref[...] + _ln(h, g[...], be[...])


def _node_body(hn_ref, s_ref, d_ref, v1n, v1a, b1, v2, b2, g, be,
               w1s, w1d, b1e, hn_out, t_ref):
    deg = d_ref[0][:, :1] + d_ref[1][:, :1]
    deg = jnp.maximum(deg, 1.0)
    agg = (s_ref[0] + s_ref[1]) / deg
    t = _dot(hn_ref[...], v1n[...]) + _dot(agg, v1a[...]) + b1[...]
    t = jnp.maximum(t, 0.0)
    h = _dot(t, v2[...]) + b2[...]
    hn = hn_ref[...] + _ln(h, g[...], be[...])
    hn_out[...] = hn
    t_ref[0] = _dot(hn, w1s[...])
    t_ref[1] = _dot(hn, w1d[...]) + b1e[...]


def _dec_body(hn_ref, est_ref, w1, b1, w2, b2, out_ref):
    t = _dot(hn_ref[...], w1[...]) + b1[...]
    t = jnp.maximum(t, 0.0)
    out_ref[...] = _dot(t, w2[...]) + b2[...] + est_ref[...]


def _wspec(r, c):
    return pl.BlockSpec((r, c), lambda i: (0, 0))


def _sc_mesh():
    return plsc.VectorSubcoreMesh(core_axis_name="c", subcore_axis_name="s",
                                  num_cores=NC, num_subcores=NS)


def _chunk_range(wid, nchunk, nworkers):
    start = wid * nchunk // nworkers
    end = (wid + 1) * nchunk // nworkers
    return start, end


K = 3


def _sc_gather_body(nchunk, t_ref, src_ref, dstn_ref, g1_ref,
                    idx_a, idx_b, rows, sem_i, sem_g, sem_w):
    wid = lax.axis_index("s") * NC + lax.axis_index("c")
    start, end = _chunk_range(wid, nchunk, NW)
    n_mine = end - start
    n_full = n_mine // K

    def group(j, carry):
        c0 = start + K * j
        off = c0 * CH
        cps = []
        for k in range(K):
            cps.append(pltpu.async_copy(
                src_ref.at[pl.ds(off + k * CH, CH)], idx_a.at[k], sem_i))
            cps.append(pltpu.async_copy(
                dstn_ref.at[pl.ds(off + k * CH, CH)], idx_b.at[k], sem_i))
        for cp in cps:
            cp.wait()
        cps = [pltpu.async_copy(t_ref.at[idx_a.at[k]],
                                rows.at[pl.ds(k * CH, CH)], sem_g)
               for k in range(K)]
        for cp in cps:
            cp.wait()
        cps = [pltpu.async_copy(t_ref.at[idx_b.at[k]],
                                rows.at[pl.ds(k * CH, CH)], sem_g, add=True)
               for k in range(K)]
        for cp in cps:
            cp.wait()
        pltpu.async_copy(rows, g1_ref.at[pl.ds(off, K * CH)], sem_w).wait()
        return carry

    lax.fori_loop(0, n_full, group, 0)

    def tail(i, carry):
        off = (start + i) * CH
        pltpu.sync_copy(src_ref.at[pl.ds(off, CH)], idx_a.at[0])
        pltpu.sync_copy(dstn_ref.at[pl.ds(off, CH)], idx_b.at[0])
        pltpu.async_copy(t_ref.at[idx_a.at[0]],
                         rows.at[pl.ds(0, CH)], sem_g).wait()
        pltpu.async_copy(t_ref.at[idx_b.at[0]],
                         rows.at[pl.ds(0, CH)], sem_g, add=True).wait()
        pltpu.sync_copy(rows.at[pl.ds(0, CH)], g1_ref.at[pl.ds(off, CH)])
        return carry

    lax.fori_loop(K * n_full, n_mine, tail, 0)


def _scatter_range(w, nchunk, he_ref, dst_ref, idx_v, rows_v, acc, sem):
    start, end = _chunk_range(w, nchunk, NW)
    n_mine = end - start
    n_full = n_mine // K

    def group(j, carry):
        c0 = start + K * j
        off = c0 * CH
        cps = [pltpu.async_copy(he_ref.at[pl.ds(off, K * CH)], rows_v, sem)]
        for k in range(K):
            cps.append(pltpu.async_copy(
                dst_ref.at[pl.ds(off + k * CH, CH)], idx_v.at[k], sem))
        for cp in cps:
            cp.wait()
        cps = [pltpu.async_copy(rows_v.at[pl.ds(k * CH, CH)],
                                acc.at[idx_v.at[k]], sem, add=True)
               for k in range(K)]
        for cp in cps:
            cp.wait()
        return carry

    lax.fori_loop(0, n_full, group, 0)

    def tail(i, carry):
        off = (start + i) * CH
        pltpu.sync_copy(dst_ref.at[pl.ds(off, CH)], idx_v.at[0])
        pltpu.sync_copy(he_ref.at[pl.ds(off, CH)], rows_v.at[pl.ds(0, CH)])
        pltpu.sync_copy(rows_v.at[pl.ds(0, CH)], acc.at[idx_v.at[0]],
                        add=True)
        return carry

    lax.fori_loop(K * n_full, n_mine, tail, 0)


def _sc_scatter_body(rows_per_tile, nchunk_h, he_ref, dst_ref, init_ref,
                     out_ref, idx_v, rows_v, acc, sem):
    cid = lax.axis_index("c")
    sid = lax.axis_index("s")
    wid = sid * NC + cid
    r0 = sid * rows_per_tile
    pltpu.sync_copy(init_ref.at[cid, pl.ds(r0, rows_per_tile)],
                    acc.at[pl.ds(r0, rows_per_tile)])
    plsc.subcore_barrier()
    _scatter_range(wid, nchunk_h, he_ref, dst_ref, idx_v, rows_v, acc, sem)
    plsc.subcore_barrier()
    pltpu.sync_copy(acc.at[pl.ds(r0, rows_per_tile)],
                    out_ref.at[cid, pl.ds(r0, rows_per_tile)])


def kernel(x, edge_index, edge_attr, estimate, params):
    n = x.shape[0]
    e = edge_index.shape[1]
    d_out = estimate.shape[1]
    d_edge = edge_attr.shape[1]
    eh = e // 2
    assert eh % CH == 0 and eh % BE == 0 and n % BN == 0
    nchunk_h = eh // CH
    rows_per_tile = (n + NS * 8 - 1) // (NS * 8) * 8
    n_pad = rows_per_tile * NS
    grid_n = n // BN
    grid_e = eh // BE

    src = edge_index[0]
    dst = edge_index[1]
    dstn = dst + n
    src_h = (src[:eh], src[eh:])
    dst_h = (dst[:eh], dst[eh:])
    dstn_h = (dstn[:eh], dstn[eh:])

    p_enc_n = params['enc_n']
    p_enc_e = params['enc_e']
    p_dec = params['dec']
    procs = params['proc']
    nlayers = len(procs)

    def r1(v):
        return v.reshape(1, -1)

    ew = []
    for lp in procs:
        w1 = lp['edge']['W1']
        ew.append(dict(w1e=w1[:H], w1s=w1[H:2 * H], w1d=w1[2 * H:],
                       b1=r1(lp['edge']['b1']), w2=lp['edge']['W2'],
                       b2=r1(lp['edge']['b2']), g=r1(lp['edge']['g']),
                       be=r1(lp['edge']['be'])))
    nw_ = []
    for lp in procs:
        w1 = lp['node']['W1']
        nw_.append(dict(v1n=w1[:H], v1a=w1[H:], b1=r1(lp['node']['b1']),
                        v2=lp['node']['W2'], b2=r1(lp['node']['b2']),
                        g=r1(lp['node']['g']), be=r1(lp['node']['be'])))

    f32 = jnp.float32

    enc_n_call = pl.pallas_call(
        _enc_node_body,
        grid=(grid_n,),
        in_specs=[
            pl.BlockSpec((BN, H), lambda i: (i, 0)),
            pl.BlockSpec((BN, d_out), lambda i: (i, 0)),
            _wspec(H, H), _wspec(d_out, H), _wspec(1, H),
            _wspec(H, H), _wspec(1, H), _wspec(1, H), _wspec(1, H),
            _wspec(H, H), _wspec(H, H), _wspec(1, H),
        ],
        out_specs=[
            pl.BlockSpec((BN, H), lambda i: (i, 0)),
            pl.BlockSpec((2, BN, H), lambda i: (0, i, 0)),
        ],
        out_shape=[
            jax.ShapeDtypeStruct((n, H), f32),
            jax.ShapeDtypeStruct((2, n, H), f32),
        ],
    )
    w1n = p_enc_n['W1']
    hn, tbl = enc_n_call(x, estimate, w1n[:H], w1n[H:], r1(p_enc_n['b1']),
                         p_enc_n['W2'], r1(p_enc_n['b2']), r1(p_enc_n['g']),
                         r1(p_enc_n['be']), ew[0]['w1s'], ew[0]['w1d'],
                         ew[0]['b1'])

    enc_e_call = pl.pallas_call(
        _enc_edge_body,
        grid=(grid_e,),
        in_specs=[
            pl.BlockSpec((BE, d_edge), lambda i: (i, 0)),
            _wspec(d_edge, H), _wspec(1, H), _wspec(H, H),
            _wspec(1, H), _wspec(1, H), _wspec(1, H),
        ],
        out_specs=pl.BlockSpec((BE, H), lambda i: (i, 0)),
        out_shape=jax.ShapeDtypeStruct((eh, H), f32),
    )
    he_h = [enc_e_call(edge_attr[i * eh:(i + 1) * eh], p_enc_e['W1'],
                       r1(p_enc_e['b1']), p_enc_e['W2'], r1(p_enc_e['b2']),
                       r1(p_enc_e['g']), r1(p_enc_e['be']))
            for i in range(2)]

    gather_call = pl.kernel(
        functools.partial(_sc_gather_body, nchunk_h),
        out_type=jax.ShapeDtypeStruct((eh, H), f32),
        mesh=_sc_mesh(),
        scratch_types=[
            pltpu.VMEM((K, CH), jnp.int32),
            pltpu.VMEM((K, CH), jnp.int32),
            pltpu.VMEM((K * CH, H), f32),
            pltpu.SemaphoreType.DMA,
            pltpu.SemaphoreType.DMA,
            pltpu.SemaphoreType.DMA,
        ],
    )

    scatter_call = pl.kernel(
        functools.partial(_sc_scatter_body, rows_per_tile, nchunk_h),
        out_type=jax.ShapeDtypeStruct((NC, n_pad, H), f32),
        mesh=_sc_mesh(),
        scratch_types=[
            pltpu.VMEM((K, CH), jnp.int32),
            pltpu.VMEM((K * CH, H), f32),
            pltpu.VMEM_SHARED((n_pad, H), f32),
            pltpu.SemaphoreType.DMA,
        ],
    )
    zeros_acc = jnp.zeros((NC, n_pad, H), f32)

    ones_h = jnp.ones((eh, H), f32)
    degp = scatter_call(ones_h, dst_h[1], scatter_call(ones_h, dst_h[0],
                                                       zeros_acc))
    degp1 = degp[:, :, :1]

    edge_call = pl.pallas_call(
        _edge_body,
        grid=(grid_e,),
        in_specs=[
            pl.BlockSpec((BE, H), lambda i: (i, 0)),
            pl.BlockSpec((BE, H), lambda i: (i, 0)),
            _wspec(H, H), _wspec(H, H), _wspec(1, H),
            _wspec(1, H), _wspec(1, H),
        ],
        out_specs=pl.BlockSpec((BE, H), lambda i: (i, 0)),
        out_shape=jax.ShapeDtypeStruct((eh, H), f32),
    )
    node_call = pl.pallas_call(
        _node_body,
        grid=(grid_n,),
        in_specs=[
            pl.BlockSpec((BN, H), lambda i: (i, 0)),
            pl.BlockSpec((2, BN, H), lambda i: (0, i, 0)),
            pl.BlockSpec((2, BN, 1), lambda i: (0, i, 0)),
            _wspec(H, H), _wspec(H, H), _wspec(1, H),
            _wspec(H, H), _wspec(1, H), _wspec(1, H), _wspec(1, H),
            _wspec(H, H), _wspec(H, H), _wspec(1, H),
        ],
        out_specs=[
            pl.BlockSpec((BN, H), lambda i: (i, 0)),
            pl.BlockSpec((2, BN, H), lambda i: (0, i, 0)),
        ],
        out_shape=[
            jax.ShapeDtypeStruct((n, H), f32),
            jax.ShapeDtypeStruct((2, n, H), f32),
        ],
    )

    zero_w = jnp.zeros((H, H), f32)
    zero_b = jnp.zeros((1, H), f32)
    for l in range(nlayers):
        tbl2 = tbl.reshape(2 * n, H)
        ewl = ew[l]
        s = zeros_acc
        for i in range(2):
            g1 = gather_call(tbl2, src_h[i], dstn_h[i])
            he_h[i] = edge_call(he_h[i], g1, ewl['w1e'], ewl['w2'],
                                ewl['b2'], ewl['g'], ewl['be'])
            s = scatter_call(he_h[i], dst_h[i], s)
        nwl = nw_[l]
        if l + 1 < nlayers:
            w1s_n, w1d_n, b1_n = (ew[l + 1]['w1s'], ew[l + 1]['w1d'],
                                  ew[l + 1]['b1'])
        else:
            w1s_n, w1d_n, b1_n = zero_w, zero_w, zero_b
        hn, tbl = node_call(hn, s, degp1, nwl['v1n'], nwl['v1a'], nwl['b1'],
                            nwl['v2'], nwl['b2'], nwl['g'], nwl['be'],
                            w1s_n, w1d_n, b1_n)

    dec_call = pl.pallas_call(
        _dec_body,
        grid=(grid_n,),
        in_specs=[
            pl.BlockSpec((BN, H), lambda i: (i, 0)),
            pl.BlockSpec((BN, d_out), lambda i: (i, 0)),
            _wspec(H, H), _wspec(1, H), _wspec(H, d_out), _wspec(1, d_out),
        ],
        out_specs=pl.BlockSpec((BN, d_out), lambda i: (i, 0)),
        out_shape=jax.ShapeDtypeStruct((n, d_out), f32),
    )
    out = dec_call(hn, estimate, p_dec['W1'], r1(p_dec['b1']),
                   p_dec['W2'], r1(p_dec['b2']))
    return out
